# baseline (speedup 1.0000x reference)
import functools

import jax
import jax.numpy as jnp
from jax import lax
from jax.experimental import pallas as pl
from jax.experimental.pallas import tpu as pltpu
from jax.experimental.pallas import tpu_sc as plsc

CHILD = 16
NODE_TOTAL = 273
BATCH = 4096
STATE_DIM = 2048
LANE = 128

_TC_BLOCK = 1024
_KC = 256


def _tc_body(state_ref, wt_ref, b_ref, aval_ref, clip_ref, probt_ref):
    def chunk(i):
        return lax.dot_general(
            state_ref[:, i * _KC:(i + 1) * _KC], wt_ref[:, i * _KC:(i + 1) * _KC],
            (((1,), (1,)), ((), ())), preferred_element_type=jnp.float32)

    acc = chunk(0)
    for i in range(1, STATE_DIM // _KC):
        acc = acc + chunk(i)
    logits = acc + b_ref[...]
    clip_ref[:, 0:CHILD] = jnp.clip(jax.nn.relu(logits), 1e-30, 1.0)

    @pl.when(pl.program_id(0) == 0)
    def _():
        a = aval_ref[...]
        p = a / jnp.sum(a, axis=0, keepdims=True)
        probt_ref[0:CHILD, :] = p[:, 0:LANE]
        probt_ref[CHILD:2 * CHILD, :] = p[:, LANE:2 * LANE]
        probt_ref[2 * CHILD:3 * CHILD, 0:NODE_TOTAL - 2 * LANE] = p[:, 2 * LANE:NODE_TOTAL]


def _tc_stage(state, Wt, b2d, aval_val, off, nrows):
    grid = nrows // _TC_BLOCK
    return pl.pallas_call(
        _tc_body,
        grid=(grid,),
        in_specs=[
            pl.BlockSpec((_TC_BLOCK, STATE_DIM), lambda i: (i + off, 0)),
            pl.BlockSpec((CHILD, STATE_DIM), lambda i: (0, 0)),
            pl.BlockSpec((1, CHILD), lambda i: (0, 0)),
            pl.BlockSpec((CHILD, NODE_TOTAL), lambda i: (0, 0)),
        ],
        out_specs=[
            pl.BlockSpec((_TC_BLOCK, LANE), lambda i: (i, 0)),
            pl.BlockSpec((3 * CHILD, LANE), lambda i: (0, 0)),
        ],
        out_shape=[
            jax.ShapeDtypeStruct((nrows, LANE), jnp.float32),
            jax.ShapeDtypeStruct((3 * CHILD, LANE), jnp.float32),
        ],
    )(state, Wt, b2d, aval_val)


def _sc_route(clipped_flat, probt_flat, leaf_id, nb):
    info = plsc.get_sparse_core_info()
    nc, ns = info.num_cores, info.num_subcores
    nw = nc * ns
    bpw = nb // nw
    groups = bpw // CHILD
    mesh = plsc.VectorSubcoreMesh(core_axis_name="c", subcore_axis_name="s")

    @functools.partial(
        pl.kernel,
        mesh=mesh,
        compiler_params=pltpu.CompilerParams(needs_layout_passes=False),
        out_type=(
            jax.ShapeDtypeStruct((nb * CHILD,), jnp.float32),
            jax.ShapeDtypeStruct((nb,), jnp.int32),
        ),
        scratch_types=[
            pltpu.VMEM((bpw * LANE,), jnp.float32),
            pltpu.VMEM((3 * CHILD * LANE,), jnp.float32),
            pltpu.VMEM((BATCH,), jnp.int32),
            pltpu.VMEM((bpw * CHILD,), jnp.float32),
            pltpu.VMEM((bpw,), jnp.int32),
            pltpu.SemaphoreType.DMA,
            pltpu.SemaphoreType.DMA,
            pltpu.SemaphoreType.DMA,
        ],
    )
    def route(clip_hbm, probt_hbm, leaf_hbm, mix_hbm, act_hbm,
              clip_v, probt_v, leaf_v, mix_v, act_v, sem0, sem1, sem2):
        wid = lax.axis_index("s") * nc + lax.axis_index("c")
        base = wid * bpw
        cp0 = pltpu.async_copy(clip_hbm.at[pl.ds(base * LANE, bpw * LANE)], clip_v, sem0)
        cp1 = pltpu.async_copy(probt_hbm, probt_v, sem1)
        cp2 = pltpu.async_copy(leaf_hbm, leaf_v, sem2)
        cp0.wait()
        cp1.wait()
        cp2.wait()

        lanes = lax.iota(jnp.int32, CHILD)
        p0vec = plsc.load_gather(probt_v, [lanes * LANE])
        p0 = [p0vec[j] for j in range(CHILD)]

        for g in range(groups):
            spad = (g * CHILD + lanes) * LANE
            soff = g * CHILD + lanes

            rows = [plsc.load_gather(clip_v, [spad + j]) for j in range(CHILD)]

            def row(j):
                return rows[j]

            m = row(0) * p0[0]
            c0 = jnp.zeros((CHILD,), jnp.int32)
            for j in range(1, CHILD):
                v = row(j) * p0[j]
                gt = v > m
                c0 = jnp.where(gt, jnp.int32(j), c0)
                m = jnp.where(gt, v, m)

            n1 = 1 + c0
            m = row(0) * plsc.load_gather(probt_v, [n1])
            c1 = jnp.zeros((CHILD,), jnp.int32)
            for j in range(1, CHILD):
                v = row(j) * plsc.load_gather(probt_v, [j * LANE + n1])
                gt = v > m
                c1 = jnp.where(gt, jnp.int32(j), c1)
                m = jnp.where(gt, v, m)

            n2 = 17 + CHILD * c0 + c1
            base2 = ((n2 >> 7) << 11) + (n2 & 127)
            mix0 = row(0) * plsc.load_gather(probt_v, [base2])
            plsc.store_scatter(mix_v, [soff], mix0)
            m = mix0
            c2 = jnp.zeros((CHILD,), jnp.int32)
            for j in range(1, CHILD):
                v = row(j) * plsc.load_gather(probt_v, [j * LANE + base2])
                plsc.store_scatter(mix_v, [j * bpw + soff], v)
                gt = v > m
                c2 = jnp.where(gt, jnp.int32(j), c2)
                m = jnp.where(gt, v, m)

            leaf_idx = CHILD * (CHILD * c0 + c1) + c2
            act = plsc.load_gather(leaf_v, [leaf_idx])
            plsc.store_scatter(act_v, [g * CHILD + lanes], act)

        outs = [pltpu.async_copy(mix_v.at[pl.ds(j * bpw, bpw)],
                                 mix_hbm.at[pl.ds(j * nb + base, bpw)], sem0)
                for j in range(CHILD)]
        cpo1 = pltpu.async_copy(act_v, act_hbm.at[pl.ds(base, bpw)], sem1)
        for cp in outs:
            cp.wait()
        cpo1.wait()

    return route(clipped_flat, probt_flat, leaf_id)


def kernel(state, W, b, aval_val, leaf_id):
    half = BATCH // 2
    b2 = b.reshape(1, CHILD)
    clip0, probt = _tc_stage(state, W.T, b2, aval_val, 0, half)
    clip1, _ = _tc_stage(state, W.T, b2, aval_val, half // _TC_BLOCK, half)
    mix0, act0 = _sc_route(clip0.reshape(-1), probt.reshape(-1), leaf_id, half)
    mix1, act1 = _sc_route(clip1.reshape(-1), probt.reshape(-1), leaf_id, half)
    mixt = jnp.concatenate([mix0.reshape(CHILD, half), mix1.reshape(CHILD, half)], axis=1)
    return mixt.T, jnp.concatenate([act0, act1])


# speedup vs baseline: 1.1962x; 1.1962x over previous
import functools

import jax
import jax.numpy as jnp
from jax import lax
from jax.experimental import pallas as pl
from jax.experimental.pallas import tpu as pltpu
from jax.experimental.pallas import tpu_sc as plsc

CHILD = 16
NODE_TOTAL = 273
BATCH = 4096
STATE_DIM = 2048
LANE = 128

_TC_BLOCK = 1024
_KC = 256


def _tc_body(state_ref, wt_ref, b_ref, aval_ref, clip_ref, probt_ref):
    def chunk(i):
        return lax.dot_general(
            state_ref[:, i * _KC:(i + 1) * _KC], wt_ref[:, i * _KC:(i + 1) * _KC],
            (((1,), (1,)), ((), ())), preferred_element_type=jnp.float32)

    acc = chunk(0)
    for i in range(1, STATE_DIM // _KC):
        acc = acc + chunk(i)
    logits = acc + b_ref[...]
    clip_ref[:, 0:CHILD] = jnp.clip(jax.nn.relu(logits), 1e-30, 1.0)

    @pl.when(pl.program_id(0) == 0)
    def _():
        a = aval_ref[...]
        p = a / jnp.sum(a, axis=0, keepdims=True)
        probt_ref[0:CHILD, :] = p[:, 0:LANE]
        probt_ref[CHILD:2 * CHILD, :] = p[:, LANE:2 * LANE]
        probt_ref[2 * CHILD:3 * CHILD, 0:NODE_TOTAL - 2 * LANE] = p[:, 2 * LANE:NODE_TOTAL]


def _tc_stage(state, Wt, b2d, aval_val):
    grid = state.shape[0] // _TC_BLOCK
    return pl.pallas_call(
        _tc_body,
        grid=(grid,),
        in_specs=[
            pl.BlockSpec((_TC_BLOCK, STATE_DIM), lambda i: (i, 0)),
            pl.BlockSpec((CHILD, STATE_DIM), lambda i: (0, 0)),
            pl.BlockSpec((1, CHILD), lambda i: (0, 0)),
            pl.BlockSpec((CHILD, NODE_TOTAL), lambda i: (0, 0)),
        ],
        out_specs=[
            pl.BlockSpec((_TC_BLOCK, LANE), lambda i: (i, 0)),
            pl.BlockSpec((3 * CHILD, LANE), lambda i: (0, 0)),
        ],
        out_shape=[
            jax.ShapeDtypeStruct((state.shape[0], LANE), jnp.float32),
            jax.ShapeDtypeStruct((3 * CHILD, LANE), jnp.float32),
        ],
    )(state, Wt, b2d, aval_val)


def _sc_route(clipped_flat, probt_flat, leaf_id):
    info = plsc.get_sparse_core_info()
    nc, ns = info.num_cores, info.num_subcores
    nw = nc * ns
    bpw = BATCH // nw
    groups = bpw // CHILD
    mesh = plsc.VectorSubcoreMesh(core_axis_name="c", subcore_axis_name="s")

    @functools.partial(
        pl.kernel,
        mesh=mesh,
        compiler_params=pltpu.CompilerParams(needs_layout_passes=False),
        out_type=(
            jax.ShapeDtypeStruct((BATCH * CHILD,), jnp.float32),
            jax.ShapeDtypeStruct((BATCH,), jnp.int32),
        ),
        scratch_types=[
            pltpu.VMEM((bpw * LANE,), jnp.float32),
            pltpu.VMEM((3 * CHILD * LANE,), jnp.float32),
            pltpu.VMEM((BATCH,), jnp.int32),
            pltpu.VMEM((bpw * CHILD,), jnp.float32),
            pltpu.VMEM((bpw,), jnp.int32),
            pltpu.SemaphoreType.DMA,
            pltpu.SemaphoreType.DMA,
            pltpu.SemaphoreType.DMA,
        ],
    )
    def route(clip_hbm, probt_hbm, leaf_hbm, mix_hbm, act_hbm,
              clip_v, probt_v, leaf_v, mix_v, act_v, sem0, sem1, sem2):
        wid = lax.axis_index("s") * nc + lax.axis_index("c")
        base = wid * bpw
        cp0 = pltpu.async_copy(clip_hbm.at[pl.ds(base * LANE, bpw * LANE)], clip_v, sem0)
        cp1 = pltpu.async_copy(probt_hbm, probt_v, sem1)
        cp2 = pltpu.async_copy(leaf_hbm, leaf_v, sem2)
        cp0.wait()
        cp1.wait()
        cp2.wait()

        lanes = lax.iota(jnp.int32, CHILD)
        p0vec = plsc.load_gather(probt_v, [lanes * LANE])
        p0 = [p0vec[j] for j in range(CHILD)]

        for g in range(groups):
            spad = (g * CHILD + lanes) * LANE
            soff = g * CHILD + lanes

            rows = [plsc.load_gather(clip_v, [spad + j]) for j in range(CHILD)]

            def row(j):
                return rows[j]

            m = row(0) * p0[0]
            c0 = jnp.zeros((CHILD,), jnp.int32)
            for j in range(1, CHILD):
                v = row(j) * p0[j]
                gt = v > m
                c0 = jnp.where(gt, jnp.int32(j), c0)
                m = jnp.where(gt, v, m)

            n1 = 1 + c0
            m = row(0) * plsc.load_gather(probt_v, [n1])
            c1 = jnp.zeros((CHILD,), jnp.int32)
            for j in range(1, CHILD):
                v = row(j) * plsc.load_gather(probt_v, [j * LANE + n1])
                gt = v > m
                c1 = jnp.where(gt, jnp.int32(j), c1)
                m = jnp.where(gt, v, m)

            n2 = 17 + CHILD * c0 + c1
            base2 = ((n2 >> 7) << 11) + (n2 & 127)
            mix0 = row(0) * plsc.load_gather(probt_v, [base2])
            plsc.store_scatter(mix_v, [soff], mix0)
            m = mix0
            c2 = jnp.zeros((CHILD,), jnp.int32)
            for j in range(1, CHILD):
                v = row(j) * plsc.load_gather(probt_v, [j * LANE + base2])
                plsc.store_scatter(mix_v, [j * bpw + soff], v)
                gt = v > m
                c2 = jnp.where(gt, jnp.int32(j), c2)
                m = jnp.where(gt, v, m)

            leaf_idx = CHILD * (CHILD * c0 + c1) + c2
            act = plsc.load_gather(leaf_v, [leaf_idx])
            plsc.store_scatter(act_v, [g * CHILD + lanes], act)

        outs = [pltpu.async_copy(mix_v.at[pl.ds(j * bpw, bpw)],
                                 mix_hbm.at[pl.ds(j * BATCH + base, bpw)], sem0)
                for j in range(CHILD)]
        cpo1 = pltpu.async_copy(act_v, act_hbm.at[pl.ds(base, bpw)], sem1)
        for cp in outs:
            cp.wait()
        cpo1.wait()

    return route(clipped_flat, probt_flat, leaf_id)


def kernel(state, W, b, aval_val, leaf_id):
    clipped, probt = _tc_stage(state, W.T, b.reshape(1, CHILD), aval_val)
    mix_flat, act = _sc_route(clipped.reshape(-1), probt.reshape(-1), leaf_id)
    return mix_flat.reshape(CHILD, BATCH).T, act
